# baseline (speedup 1.0000x reference)
import functools

import jax
import jax.numpy as jnp
from jax import lax
from jax.experimental import pallas as pl
from jax.experimental.pallas import tpu as pltpu
from jax.experimental.pallas import tpu_sc as plsc

VOCAB = 1000000
D = 64
SEQ = 200
B = 4096

NC, NS, L = 2, 16, 16
NW = NC * NS
N = B * SEQ
ROWS_PER_W = N // NW
CB = 4
CR = CB * SEQ
G = 100
NSTREAM = CR // G
CHUNKS = ROWS_PER_W // CR


def _body(idx_hbm, table_hbm, pos_hbm, out_hbm, idx_v, rows_v, pos_v, gsem):
    c = lax.axis_index("c")
    s = lax.axis_index("s")
    wid = s * NC + c
    base = wid * ROWS_PER_W

    pltpu.sync_copy(pos_hbm, pos_v)

    def chunk(g, carry):
        r0 = pl.multiple_of(base + g * CR, CR)
        blk0 = pl.multiple_of(r0 // G, NSTREAM)
        pltpu.sync_copy(idx_hbm.at[pl.ds(blk0, NSTREAM)], idx_v)
        copies = []
        for j in range(NSTREAM):
            copies.append(
                pltpu.async_copy(table_hbm.at[idx_v.at[j]],
                                 rows_v.at[pl.ds(j * G, G)], gsem))
        for cp in copies:
            cp.wait()

        def posrow(p, cc):
            for d2 in range(D // L):
                v = pos_v[pl.ds(p * D + d2 * L, L)]
                for b in range(CB):
                    plsc.addupdate(rows_v.at[b * SEQ + p, pl.ds(d2 * L, L)], v)
            return cc

        lax.fori_loop(0, SEQ, posrow, 0)
        pltpu.sync_copy(rows_v, out_hbm.at[pl.ds(r0, CR)])
        return carry

    lax.fori_loop(0, CHUNKS, chunk, 0)


@jax.jit
def kernel(input_idx, word_table, pos_table):
    idx2 = input_idx.reshape(N // G, G).astype(jnp.int32)
    pos_flat = pos_table.reshape(-1)
    mesh = plsc.VectorSubcoreMesh(core_axis_name="c", subcore_axis_name="s")
    out = pl.kernel(
        _body,
        out_type=jax.ShapeDtypeStruct((N, D), jnp.float32),
        mesh=mesh,
        compiler_params=pltpu.CompilerParams(use_tc_tiling_on_sc=False),
        scratch_types=[
            pltpu.VMEM((NSTREAM, G), jnp.int32),
            pltpu.VMEM((CR, D), jnp.float32),
            pltpu.VMEM((SEQ * D,), jnp.float32),
            pltpu.SemaphoreType.DMA,
        ],
    )(idx2, word_table, pos_flat)
    return out.reshape(B, SEQ, D)


# speedup vs baseline: 1.0670x; 1.0670x over previous
import jax
import jax.numpy as jnp
from jax import lax
from jax.experimental import pallas as pl
from jax.experimental.pallas import tpu as pltpu
from jax.experimental.pallas import tpu_sc as plsc

VOCAB = 1000000
D = 64
SEQ = 200
B = 4096

NC, NS, L = 2, 16, 16
NW = NC * NS
N = B * SEQ
ROWS_PER_W = N // NW
CR = SEQ
G = 100
NSTREAM = CR // G
CHUNKS = ROWS_PER_W // CR
NBUF = 2
IBLKS = ROWS_PER_W // G


def _body(idx_hbm, table_hbm, pos_hbm, out_hbm,
          idx_v, rows_in, rows_out, pos_v, gsem0, gsem1, osem0, osem1):
    c = lax.axis_index("c")
    s = lax.axis_index("s")
    wid = s * NC + c
    base = wid * ROWS_PER_W
    iblk = pl.multiple_of(wid * IBLKS, 8)

    pltpu.sync_copy(pos_hbm, pos_v)
    pltpu.sync_copy(idx_hbm.at[pl.ds(iblk, IBLKS)], idx_v)

    gsems = (gsem0, gsem1)
    osems = (osem0, osem1)

    def gather_descs(gg, b, sem):
        return [pltpu.make_async_copy(
                    table_hbm.at[idx_v.at[gg * NSTREAM + j]],
                    rows_in.at[b, pl.ds(j * G, G)], sem)
                for j in range(NSTREAM)]

    def out_desc(gg, b, sem):
        r0 = pl.multiple_of(base + gg * CR, 8)
        return pltpu.make_async_copy(rows_out.at[b], out_hbm.at[pl.ds(r0, CR)], sem)

    for b in range(NBUF):
        for d_ in gather_descs(b, b, gsems[b]):
            d_.start()

    @pl.loop(0, CHUNKS, step=NBUF)
    def chunk(g):
        for b in range(NBUF):
            gg = g + b
            for d_ in gather_descs(gg, b, gsems[b]):
                d_.wait()

            @pl.when(gg >= NBUF)
            def _():
                out_desc(gg - NBUF, b, osems[b]).wait()

            @pl.loop(0, SEQ)
            def posrow(p):
                for d2 in range(D // L):
                    v = pos_v[pl.ds(p * D + d2 * L, L)]
                    rows_out[b, p, pl.ds(d2 * L, L)] = rows_in[b, p, pl.ds(d2 * L, L)] + v

            @pl.when(gg + NBUF < CHUNKS)
            def _():
                for d_ in gather_descs(gg + NBUF, b, gsems[b]):
                    d_.start()

            out_desc(gg, b, osems[b]).start()

    for b in range(NBUF):
        out_desc(CHUNKS - NBUF + b, b, osems[b]).wait()


@jax.jit
def kernel(input_idx, word_table, pos_table):
    idx2 = input_idx.reshape(N // G, G).astype(jnp.int32)
    pos_flat = pos_table.reshape(-1)
    mesh = plsc.VectorSubcoreMesh(core_axis_name="c", subcore_axis_name="s")
    out = pl.kernel(
        _body,
        out_type=jax.ShapeDtypeStruct((N, D), jnp.float32),
        mesh=mesh,
        compiler_params=pltpu.CompilerParams(use_tc_tiling_on_sc=False),
        scratch_types=[
            pltpu.VMEM((IBLKS, G), jnp.int32),
            pltpu.VMEM((NBUF, CR, D), jnp.float32),
            pltpu.VMEM((NBUF, CR, D), jnp.float32),
            pltpu.VMEM((SEQ * D,), jnp.float32),
            pltpu.SemaphoreType.DMA,
            pltpu.SemaphoreType.DMA,
            pltpu.SemaphoreType.DMA,
            pltpu.SemaphoreType.DMA,
        ],
    )(idx2, word_table, pos_flat)
    return out.reshape(B, SEQ, D)


# speedup vs baseline: 1.2296x; 1.1524x over previous
import jax
import jax.numpy as jnp
from jax import lax
from jax.experimental import pallas as pl
from jax.experimental.pallas import tpu as pltpu
from jax.experimental.pallas import tpu_sc as plsc

VOCAB = 1000000
D = 64
SEQ = 200
B = 4096
DP = 128

NC, NS, L = 2, 16, 16
NW = NC * NS
N = B * SEQ
ROWS_PER_W = N // NW
G = 128
CR = 256
NSTREAM = CR // G
CHUNKS = ROWS_PER_W // CR
NBUF = 2
IBLKS = ROWS_PER_W // G


def _body(idx_hbm, table_hbm, pos_hbm, out_hbm,
          idx_v, rows, pos_v, gsem0, gsem1, osem0, osem1):
    c = lax.axis_index("c")
    s = lax.axis_index("s")
    wid = s * NC + c
    base = wid * ROWS_PER_W
    iblk = pl.multiple_of(wid * IBLKS, 8)

    pltpu.sync_copy(pos_hbm, pos_v)
    pltpu.sync_copy(idx_hbm.at[pl.ds(iblk, IBLKS)], idx_v)

    gsems = (gsem0, gsem1)
    osems = (osem0, osem1)

    def gather_descs(gg, b, sem):
        return [pltpu.make_async_copy(
                    table_hbm.at[idx_v.at[gg * NSTREAM + j]],
                    rows.at[b, pl.ds(j * G, G)], sem)
                for j in range(NSTREAM)]

    def out_desc(gg, b, sem):
        r0 = pl.multiple_of(base + gg * CR, 8)
        return pltpu.make_async_copy(rows.at[b], out_hbm.at[pl.ds(r0, CR)], sem)

    for b in range(NBUF):
        for d_ in gather_descs(b, b, gsems[b]):
            d_.start()

    @pl.loop(0, CHUNKS, step=NBUF)
    def chunk(g):
        for b in range(NBUF):
            gg = g + b
            for d_ in gather_descs(gg, b, gsems[b]):
                d_.wait()

            @pl.when(gg >= NBUF)
            def _():
                out_desc(gg - NBUF, b, osems[b]).wait()

            p0 = lax.rem(gg * CR, SEQ)

            @pl.loop(0, CR)
            def posrow(k):
                p = p0 + k
                p = lax.select(p >= 2 * SEQ, p - 2 * SEQ,
                               lax.select(p >= SEQ, p - SEQ, p))
                for d2 in range(D // L):
                    v = pos_v[pl.ds(p * D + d2 * L, L)]
                    plsc.addupdate(rows.at[b, k, pl.ds(d2 * L, L)], v)

            @pl.when(gg + NBUF < CHUNKS)
            def _():
                for d_ in gather_descs(gg + NBUF, b, gsems[b]):
                    d_.start()

            out_desc(gg, b, osems[b]).start()

    for b in range(NBUF):
        out_desc(CHUNKS - NBUF + b, b, osems[b]).wait()


@jax.jit
def kernel(input_idx, word_table, pos_table):
    idx2 = input_idx.reshape(N // G, G).astype(jnp.int32)
    tablep = jnp.pad(word_table, ((0, 0), (0, DP - D)))
    pos_flat = pos_table.reshape(-1)
    mesh = plsc.VectorSubcoreMesh(core_axis_name="c", subcore_axis_name="s")
    out = pl.kernel(
        _body,
        out_type=jax.ShapeDtypeStruct((N, DP), jnp.float32),
        mesh=mesh,
        compiler_params=pltpu.CompilerParams(use_tc_tiling_on_sc=False),
        scratch_types=[
            pltpu.VMEM((IBLKS, G), jnp.int32),
            pltpu.VMEM((NBUF, CR, DP), jnp.float32),
            pltpu.VMEM((SEQ * D,), jnp.float32),
            pltpu.SemaphoreType.DMA,
            pltpu.SemaphoreType.DMA,
            pltpu.SemaphoreType.DMA,
            pltpu.SemaphoreType.DMA,
        ],
    )(idx2, tablep, pos_flat)
    return out[:, :D].reshape(B, SEQ, D)
